# baseline (speedup 1.0000x reference)
import functools

import jax
import jax.numpy as jnp
from jax import lax
from jax.experimental import pallas as pl
from jax.experimental.pallas import tpu as pltpu
from jax.experimental.pallas import tpu_sc as plsc

_NC, _NS = 2, 16
_NW = _NC * _NS


def _sc_gather(table, idx):
    _, d = table.shape
    b = idx.shape[0]
    b_per_w = b // _NW
    mesh = plsc.VectorSubcoreMesh(core_axis_name="c", subcore_axis_name="s")

    @functools.partial(
        pl.kernel,
        mesh=mesh,
        out_type=jax.ShapeDtypeStruct((b, d), jnp.float32),
        compiler_params=pltpu.CompilerParams(use_tc_tiling_on_sc=False),
        scratch_types=[
            pltpu.VMEM((b_per_w,), jnp.int32),
            pltpu.VMEM((b_per_w, d), jnp.float32),
            pltpu.SemaphoreType.DMA,
        ],
    )
    def gather_kernel(table_hbm, idx_hbm, out_hbm, idx_v, rows_v, sem):
        wid = lax.axis_index("s") * _NC + lax.axis_index("c")
        base = wid * b_per_w
        pltpu.sync_copy(idx_hbm.at[pl.ds(base, b_per_w)], idx_v)
        pltpu.async_copy(table_hbm.at[idx_v], rows_v, sem).wait()
        pltpu.sync_copy(rows_v, out_hbm.at[pl.ds(base, b_per_w)])

    return gather_kernel(table, idx)


def _head_matmul(x, w_head, v_blk=1024):
    t, d = x.shape
    v = w_head.shape[0]
    nv = pl.cdiv(v, v_blk)

    def body(x_ref, w_ref, o_ref):
        o_ref[...] = lax.dot_general(
            x_ref[...], w_ref[...],
            dimension_numbers=(((1,), (1,)), ((), ())),
            preferred_element_type=jnp.float32,
        )

    return pl.pallas_call(
        body,
        grid=(nv,),
        in_specs=[
            pl.BlockSpec((t, d), lambda j: (0, 0)),
            pl.BlockSpec((v_blk, d), lambda j: (j, 0)),
        ],
        out_specs=pl.BlockSpec((t, v_blk), lambda j: (0, j)),
        out_shape=jax.ShapeDtypeStruct((t, v), jnp.float32),
    )(x, w_head)


def kernel(input_ids, W_emb, W_head):
    b, t = input_ids.shape
    idx = input_ids.reshape(-1).astype(jnp.int32)
    x = _sc_gather(W_emb, idx)
    logits = _head_matmul(x, W_head)
    return logits.reshape(b, t, W_head.shape[0])
